# baseline (speedup 1.0000x reference)
import functools

import jax
import jax.numpy as jnp
import numpy as np
from jax import lax
from jax.experimental import pallas as pl
from jax.experimental.pallas import tpu as pltpu
from jax.experimental.pallas import tpu_sc as plsc

N = 10000
E = 320000
F_IN = 128
HID = 64
F_OUT = 128
SH0 = 1.0 / (2.0 * np.sqrt(np.pi))
ALPHA = 1.0 / np.sqrt(F_IN * 1.0)
SCALE = SH0 * ALPHA

NC = 2
NS = 16
NW = NC * NS
L = 16

CHUNK = 128
NBUF = 4
CH = 80
NG = CH // NBUF
EPT = CH * CHUNK
E_PAD = NW * EPT
N_PAD = 10016
RPT = 640
PAD_E = E_PAD - E


def _z_body(x_ref, wtp_ref, w1_ref, ei_ref, z_ref, row_ref, col_ref):
    wa = jnp.dot(wtp_ref[...], w1_ref[...], preferred_element_type=jnp.float32)
    z_ref[...] = jnp.dot(x_ref[...], wa,
                         preferred_element_type=jnp.float32) * SCALE
    ei = ei_ref[...]
    row_ref[pl.ds(0, E)] = ei[0]
    col_ref[pl.ds(0, E)] = ei[1]
    padi = lax.broadcasted_iota(jnp.int32, (PAD_E,), 0)
    row_ref[pl.ds(E, PAD_E)] = N + padi % (N_PAD - N)
    col_ref[pl.ds(E, PAD_E)] = padi % N


def _z_matmul(x, W_tp, W1, edge_index):
    return pl.pallas_call(
        _z_body,
        out_shape=(
            jax.ShapeDtypeStruct((N, HID), jnp.float32),
            jax.ShapeDtypeStruct((E_PAD,), jnp.int32),
            jax.ShapeDtypeStruct((E_PAD,), jnp.int32),
        ),
    )(x, W_tp, W1, edge_index)


def _sc_body(z_hbm, col_hbm, row_hbm, acc_out,
             colv, rowv, ga0, ga1, ga2, ga3, gb0, gb1, gb2, gb3,
             rma, rmb, onesb, zb, zb16, tcnt, accs, cnts,
             gsem, ssem, csem, csem2, msem):
    cid = lax.axis_index("c")
    sid = lax.axis_index("s")
    wid = cid * NS + sid
    mwid = (1 - cid) * NS + sid
    bufs_a = [ga0, ga1, ga2, ga3]
    bufs_b = [gb0, gb1, gb2, gb3]

    zeros16 = jnp.zeros((L,), jnp.float32)
    ones16 = jnp.ones((L,), jnp.float32)

    def _fill_row(i, c):
        for j in range(HID // L):
            zb[i, pl.ds(j * L, L)] = zeros16
        return c
    lax.fori_loop(0, 32, _fill_row, 0)

    def _fill_ones(i, c):
        onesb[pl.ds(i * L, L)] = ones16
        return c
    lax.fori_loop(0, CHUNK // L, _fill_ones, 0)

    def _fill_z16(i, c):
        zb16[pl.ds(i * L, L)] = zeros16
        return c
    lax.fori_loop(0, 512 // L, _fill_z16, 0)

    r0 = jnp.minimum(sid * RPT, N_PAD - RPT)
    zcopies = []
    off = 0
    while off < RPT:
        sz = min(32, RPT - off)
        zcopies.append((zb.at[pl.ds(0, sz)], accs.at[pl.ds(r0 + off, sz)]))
        off += sz
    zcopies.append((zb16.at[pl.ds(0, 512)], cnts.at[pl.ds(r0, 512)]))
    zcopies.append((zb16.at[pl.ds(0, RPT - 512)],
                    cnts.at[pl.ds(r0 + 512, RPT - 512)]))
    for src, dst in zcopies:
        pltpu.async_copy(src, dst, ssem)
    pltpu.async_copy(col_hbm.at[wid], colv, gsem)
    pltpu.async_copy(row_hbm.at[wid], rowv, gsem)
    pltpu.make_async_copy(col_hbm.at[wid], colv, gsem).wait()
    pltpu.make_async_copy(row_hbm.at[wid], rowv, gsem).wait()

    def _fire_gathers(g, bufs, rm):
        for b in range(NBUF):
            pltpu.async_copy(z_hbm.at[colv.at[g * NBUF + b]], bufs[b], gsem)
        pltpu.async_copy(row_hbm.at[mwid, pl.ds(g * NBUF, NBUF)], rm, msem)

    def _group(g, cur, nxt, rmcur, rmnxt):
        @pl.when(g >= 1)
        def _():
            for b in range(NBUF):
                pltpu.make_async_copy(
                    onesb, cnts.at[rmnxt.at[b]], csem2).wait()

        @pl.when(g + 1 < NG)
        def _():
            _fire_gathers(g + 1, nxt, rmnxt)
        for b in range(NBUF):
            pltpu.make_async_copy(
                z_hbm.at[colv.at[g * NBUF + b]], cur[b], gsem).wait()
        pltpu.make_async_copy(
            row_hbm.at[mwid, pl.ds(g * NBUF, NBUF)], rmcur, msem).wait()
        for b in range(NBUF):
            j = g * NBUF + b
            pltpu.async_copy(cur[b], accs.at[rowv.at[j]], ssem, add=True)
            pltpu.async_copy(onesb, cnts.at[rowv.at[j]], csem, add=True)
            pltpu.async_copy(onesb, cnts.at[rmcur.at[b]], csem2, add=True)
        for b in range(NBUF):
            j = g * NBUF + b
            pltpu.make_async_copy(cur[b], accs.at[rowv.at[j]], ssem).wait()

    _fire_gathers(0, bufs_a, rma)
    for src, dst in zcopies:
        pltpu.make_async_copy(src, dst, ssem).wait()
    plsc.subcore_barrier()

    def _pair(p, c):
        _group(p * 2, bufs_a, bufs_b, rma, rmb)
        _group(p * 2 + 1, bufs_b, bufs_a, rmb, rma)
        return c
    lax.fori_loop(0, NG // 2, _pair, 0)

    def _drain_cnt(j, c):
        pltpu.make_async_copy(onesb, cnts.at[rowv.at[j]], csem).wait()
        return c
    lax.fori_loop(0, CH, _drain_cnt, 0)
    for b in range(NBUF):
        pltpu.make_async_copy(onesb, cnts.at[rmb.at[b]], csem2).wait()
    plsc.subcore_barrier()

    pltpu.sync_copy(cnts.at[pl.ds(r0, RPT)], tcnt)

    def _recip(i, c):
        v = tcnt[pl.ds(i * L, L)]
        tcnt[pl.ds(i * L, L)] = 1.0 / jnp.maximum(v, 1.0)
        return c
    lax.fori_loop(0, RPT // L, _recip, 0)

    bidx = [jnp.full((L, 1), r, jnp.int32) for r in range(L)]
    dnums = lax.GatherDimensionNumbers(
        offset_dims=(), collapsed_slice_dims=(0,), start_index_map=(0,))

    def _bcast(v16, r):
        return lax.gather(v16, bidx[r], dnums, slice_sizes=(1,),
                          mode=lax.GatherScatterMode.PROMISE_IN_BOUNDS)

    NCH = RPT // CHUNK
    sb = bufs_a

    def _in_copy(k, buf):
        return pltpu.make_async_copy(
            accs.at[pl.ds(r0 + k * CHUNK, CHUNK)], buf, gsem)

    def _out_copy(k, buf):
        return pltpu.make_async_copy(
            buf, acc_out.at[cid, pl.ds(r0 + k * CHUNK, CHUNK)], ssem)

    for k in range(min(4, NCH)):
        _in_copy(k, sb[k % 4]).start()
    for k in range(NCH):
        buf = sb[k % 4]
        if k >= 4:
            _out_copy(k - 4, buf).wait()
            _in_copy(k, buf).start()
        _in_copy(k, buf).wait()

        def _scale_group(g, c2, _k=k, _buf=buf):
            v16 = tcnt[pl.ds(_k * CHUNK + g * L, L)]
            for r in range(L):
                rv = _bcast(v16, r)
                row = g * L + r
                for cg in range(HID // L):
                    sl = pl.ds(cg * L, L)
                    _buf[row, sl] = _buf[row, sl] * rv
            return c2
        lax.fori_loop(0, CHUNK // L, _scale_group, 0)
        _out_copy(k, buf).start()
    for k in range(max(0, NCH - 4), NCH):
        _out_copy(k, sb[k % 4]).wait()


_sc_out_type = jax.ShapeDtypeStruct((NC, N_PAD, HID), jnp.float32)


_sc_segment = functools.partial(
    pl.kernel,
    mesh=plsc.VectorSubcoreMesh(core_axis_name="c", subcore_axis_name="s"),
    compiler_params=pltpu.CompilerParams(use_tc_tiling_on_sc=False),
    out_type=_sc_out_type,
    scratch_types=[
        pltpu.VMEM((CH, CHUNK), jnp.int32),
        pltpu.VMEM((CH, CHUNK), jnp.int32),
    ] + [pltpu.VMEM((CHUNK, HID), jnp.float32) for _ in range(2 * NBUF)]
    + [
        pltpu.VMEM((NBUF, CHUNK), jnp.int32),
        pltpu.VMEM((NBUF, CHUNK), jnp.int32),
        pltpu.VMEM((CHUNK,), jnp.float32),
        pltpu.VMEM((32, HID), jnp.float32),
        pltpu.VMEM((512,), jnp.float32),
        pltpu.VMEM((RPT,), jnp.float32),
        pltpu.VMEM_SHARED((N_PAD, HID), jnp.float32),
        pltpu.VMEM_SHARED((N_PAD,), jnp.float32),
        pltpu.SemaphoreType.DMA,
        pltpu.SemaphoreType.DMA,
        pltpu.SemaphoreType.DMA,
        pltpu.SemaphoreType.DMA,
        pltpu.SemaphoreType.DMA,
    ],
)(_sc_body)


def _out_body(acc_ref, b1_ref, w2_ref, b2_ref, o_ref):
    aggr = acc_ref[0, :N] + acc_ref[1, :N]
    h = jnp.maximum(aggr + b1_ref[...], 0.0)
    o_ref[...] = jnp.dot(h, w2_ref[...],
                         preferred_element_type=jnp.float32) + b2_ref[...]


def _out_mlp(acc, b1, W2, b2):
    return pl.pallas_call(
        _out_body,
        out_shape=jax.ShapeDtypeStruct((N, F_OUT), jnp.float32),
    )(acc, b1.reshape(1, HID), W2, b2.reshape(1, F_OUT))


def kernel(x, edge_index, pos, W_tp, W1, b1, W2, b2):
    del pos
    z, row_p, col_p = _z_matmul(x, W_tp, W1, edge_index)
    row_p = row_p.reshape(NW, CH, CHUNK)
    col_p = col_p.reshape(NW, CH, CHUNK)
    aggr = _sc_segment(z, col_p, row_p)
    return _out_mlp(aggr, b1, W2, b2)


# speedup vs baseline: 1.0027x; 1.0027x over previous
import functools

import jax
import jax.numpy as jnp
import numpy as np
from jax import lax
from jax.experimental import pallas as pl
from jax.experimental.pallas import tpu as pltpu
from jax.experimental.pallas import tpu_sc as plsc

N = 10000
E = 320000
F_IN = 128
HID = 64
F_OUT = 128
SH0 = 1.0 / (2.0 * np.sqrt(np.pi))
ALPHA = 1.0 / np.sqrt(F_IN * 1.0)
SCALE = SH0 * ALPHA

NC = 2
NS = 16
NW = NC * NS
L = 16

CHUNK = 128
NBUF = 4
CH = 80
NG = CH // NBUF
EPT = CH * CHUNK
E_PAD = NW * EPT
N_PAD = 10016
RPT = 640
PAD_E = E_PAD - E


def _z_body(x_ref, wtp_ref, w1_ref, ei_ref, z_ref, row_ref, col_ref):
    wa = jnp.dot(wtp_ref[...], w1_ref[...], preferred_element_type=jnp.float32)
    z_ref[...] = jnp.dot(x_ref[...], wa,
                         preferred_element_type=jnp.float32) * SCALE
    ei = ei_ref[...]
    row_ref[pl.ds(0, E)] = ei[0]
    col_ref[pl.ds(0, E)] = ei[1]
    padi = lax.broadcasted_iota(jnp.int32, (PAD_E,), 0)
    row_ref[pl.ds(E, PAD_E)] = N + padi % (N_PAD - N)
    col_ref[pl.ds(E, PAD_E)] = padi % N


def _z_matmul(x, W_tp, W1, edge_index):
    return pl.pallas_call(
        _z_body,
        out_shape=(
            jax.ShapeDtypeStruct((N, HID), jnp.float32),
            jax.ShapeDtypeStruct((E_PAD,), jnp.int32),
            jax.ShapeDtypeStruct((E_PAD,), jnp.int32),
        ),
    )(x, W_tp, W1, edge_index)


def _sc_body(z_hbm, col_hbm, row_hbm, acc_out,
             colv, rowv, ga0, ga1, ga2, ga3, gb0, gb1, gb2, gb3,
             rma, rmb, onesb, zb, zb16, tcnt, accs, cnts,
             gsem, ssem, csem, csem2, msem):
    cid = lax.axis_index("c")
    sid = lax.axis_index("s")
    wid = cid * NS + sid
    mwid = (1 - cid) * NS + sid
    bufs_a = [ga0, ga1, ga2, ga3]
    bufs_b = [gb0, gb1, gb2, gb3]

    zeros16 = jnp.zeros((L,), jnp.float32)
    ones16 = jnp.ones((L,), jnp.float32)

    def _fill_row(i, c):
        for j in range(HID // L):
            zb[i, pl.ds(j * L, L)] = zeros16
        return c
    lax.fori_loop(0, 32, _fill_row, 0)

    def _fill_ones(i, c):
        onesb[pl.ds(i * L, L)] = ones16
        return c
    lax.fori_loop(0, CHUNK // L, _fill_ones, 0)

    def _fill_z16(i, c):
        zb16[pl.ds(i * L, L)] = zeros16
        return c
    lax.fori_loop(0, 512 // L, _fill_z16, 0)

    r0 = jnp.minimum(sid * RPT, N_PAD - RPT)
    zcopies = []
    off = 0
    while off < RPT:
        sz = min(32, RPT - off)
        zcopies.append((zb.at[pl.ds(0, sz)], accs.at[pl.ds(r0 + off, sz)]))
        off += sz
    zcopies.append((zb16.at[pl.ds(0, 512)], cnts.at[pl.ds(r0, 512)]))
    zcopies.append((zb16.at[pl.ds(0, RPT - 512)],
                    cnts.at[pl.ds(r0 + 512, RPT - 512)]))
    for src, dst in zcopies:
        pltpu.async_copy(src, dst, ssem)
    pltpu.async_copy(col_hbm.at[wid], colv, gsem)
    pltpu.async_copy(row_hbm.at[wid], rowv, gsem)
    pltpu.make_async_copy(col_hbm.at[wid], colv, gsem).wait()
    pltpu.make_async_copy(row_hbm.at[wid], rowv, gsem).wait()

    def _fire_gathers(g, bufs, rm):
        for b in range(NBUF):
            pltpu.async_copy(z_hbm.at[colv.at[g * NBUF + b]], bufs[b], gsem)
        pltpu.async_copy(row_hbm.at[mwid, pl.ds(g * NBUF, NBUF)], rm, msem)

    def _group(g, cur, nxt, rmcur, rmnxt):
        @pl.when(g >= 1)
        def _():
            for b in range(NBUF):
                pltpu.make_async_copy(
                    nxt[b], accs.at[rowv.at[(g - 1) * NBUF + b]],
                    ssem).wait()
                pltpu.make_async_copy(
                    onesb, cnts.at[rmnxt.at[b]], csem2).wait()

        @pl.when(g + 1 < NG)
        def _():
            _fire_gathers(g + 1, nxt, rmnxt)
        for b in range(NBUF):
            pltpu.make_async_copy(
                z_hbm.at[colv.at[g * NBUF + b]], cur[b], gsem).wait()
        pltpu.make_async_copy(
            row_hbm.at[mwid, pl.ds(g * NBUF, NBUF)], rmcur, msem).wait()
        for b in range(NBUF):
            j = g * NBUF + b
            pltpu.async_copy(cur[b], accs.at[rowv.at[j]], ssem, add=True)
            pltpu.async_copy(onesb, cnts.at[rowv.at[j]], csem, add=True)
            pltpu.async_copy(onesb, cnts.at[rmcur.at[b]], csem2, add=True)

    _fire_gathers(0, bufs_a, rma)
    for src, dst in zcopies:
        pltpu.make_async_copy(src, dst, ssem).wait()
    plsc.subcore_barrier()

    def _pair(p, c):
        _group(p * 2, bufs_a, bufs_b, rma, rmb)
        _group(p * 2 + 1, bufs_b, bufs_a, rmb, rma)
        return c
    lax.fori_loop(0, NG // 2, _pair, 0)

    def _drain_cnt(j, c):
        pltpu.make_async_copy(onesb, cnts.at[rowv.at[j]], csem).wait()
        return c
    lax.fori_loop(0, CH, _drain_cnt, 0)
    for b in range(NBUF):
        pltpu.make_async_copy(
            bufs_b[b], accs.at[rowv.at[(NG - 1) * NBUF + b]], ssem).wait()
        pltpu.make_async_copy(onesb, cnts.at[rmb.at[b]], csem2).wait()
    plsc.subcore_barrier()

    pltpu.sync_copy(cnts.at[pl.ds(r0, RPT)], tcnt)

    def _recip(i, c):
        v = tcnt[pl.ds(i * L, L)]
        tcnt[pl.ds(i * L, L)] = 1.0 / jnp.maximum(v, 1.0)
        return c
    lax.fori_loop(0, RPT // L, _recip, 0)

    bidx = [jnp.full((L, 1), r, jnp.int32) for r in range(L)]
    dnums = lax.GatherDimensionNumbers(
        offset_dims=(), collapsed_slice_dims=(0,), start_index_map=(0,))

    def _bcast(v16, r):
        return lax.gather(v16, bidx[r], dnums, slice_sizes=(1,),
                          mode=lax.GatherScatterMode.PROMISE_IN_BOUNDS)

    NCH = RPT // CHUNK
    sb = bufs_a

    def _in_copy(k, buf):
        return pltpu.make_async_copy(
            accs.at[pl.ds(r0 + k * CHUNK, CHUNK)], buf, gsem)

    def _out_copy(k, buf):
        return pltpu.make_async_copy(
            buf, acc_out.at[cid, pl.ds(r0 + k * CHUNK, CHUNK)], ssem)

    for k in range(min(4, NCH)):
        _in_copy(k, sb[k % 4]).start()
    for k in range(NCH):
        buf = sb[k % 4]
        if k >= 4:
            _out_copy(k - 4, buf).wait()
            _in_copy(k, buf).start()
        _in_copy(k, buf).wait()

        def _scale_group(g, c2, _k=k, _buf=buf):
            v16 = tcnt[pl.ds(_k * CHUNK + g * L, L)]
            for r in range(L):
                rv = _bcast(v16, r)
                row = g * L + r
                for cg in range(HID // L):
                    sl = pl.ds(cg * L, L)
                    _buf[row, sl] = _buf[row, sl] * rv
            return c2
        lax.fori_loop(0, CHUNK // L, _scale_group, 0)
        _out_copy(k, buf).start()
    for k in range(max(0, NCH - 4), NCH):
        _out_copy(k, sb[k % 4]).wait()


_sc_out_type = jax.ShapeDtypeStruct((NC, N_PAD, HID), jnp.float32)


_sc_segment = functools.partial(
    pl.kernel,
    mesh=plsc.VectorSubcoreMesh(core_axis_name="c", subcore_axis_name="s"),
    compiler_params=pltpu.CompilerParams(use_tc_tiling_on_sc=False),
    out_type=_sc_out_type,
    scratch_types=[
        pltpu.VMEM((CH, CHUNK), jnp.int32),
        pltpu.VMEM((CH, CHUNK), jnp.int32),
    ] + [pltpu.VMEM((CHUNK, HID), jnp.float32) for _ in range(2 * NBUF)]
    + [
        pltpu.VMEM((NBUF, CHUNK), jnp.int32),
        pltpu.VMEM((NBUF, CHUNK), jnp.int32),
        pltpu.VMEM((CHUNK,), jnp.float32),
        pltpu.VMEM((32, HID), jnp.float32),
        pltpu.VMEM((512,), jnp.float32),
        pltpu.VMEM((RPT,), jnp.float32),
        pltpu.VMEM_SHARED((N_PAD, HID), jnp.float32),
        pltpu.VMEM_SHARED((N_PAD,), jnp.float32),
        pltpu.SemaphoreType.DMA,
        pltpu.SemaphoreType.DMA,
        pltpu.SemaphoreType.DMA,
        pltpu.SemaphoreType.DMA,
        pltpu.SemaphoreType.DMA,
    ],
)(_sc_body)


def _out_body(acc_ref, b1_ref, w2_ref, b2_ref, o_ref):
    aggr = acc_ref[0, :N] + acc_ref[1, :N]
    h = jnp.maximum(aggr + b1_ref[...], 0.0)
    o_ref[...] = jnp.dot(h, w2_ref[...],
                         preferred_element_type=jnp.float32) + b2_ref[...]


def _out_mlp(acc, b1, W2, b2):
    return pl.pallas_call(
        _out_body,
        out_shape=jax.ShapeDtypeStruct((N, F_OUT), jnp.float32),
    )(acc, b1.reshape(1, HID), W2, b2.reshape(1, F_OUT))


def kernel(x, edge_index, pos, W_tp, W1, b1, W2, b2):
    del pos
    z, row_p, col_p = _z_matmul(x, W_tp, W1, edge_index)
    row_p = row_p.reshape(NW, CH, CHUNK)
    col_p = col_p.reshape(NW, CH, CHUNK)
    aggr = _sc_segment(z, col_p, row_p)
    return _out_mlp(aggr, b1, W2, b2)
